# baseline (speedup 1.0000x reference)
import functools

import jax
import jax.numpy as jnp
from jax import lax
from jax.experimental import pallas as pl
from jax.experimental.pallas import tpu as pltpu
from jax.experimental.pallas import tpu_sc as plsc

NUM_EMB = 100000
EMB_DIM = 32
BATCH = 16384

_info = plsc.get_sparse_core_info()
_NC, _NS = _info.num_cores, _info.num_subcores
_NW = _NC * _NS
_B_PER_W = BATCH // _NW


@functools.partial(
    pl.kernel,
    mesh=plsc.VectorSubcoreMesh(core_axis_name="c", subcore_axis_name="s"),
    out_type=jax.ShapeDtypeStruct((BATCH, EMB_DIM), jnp.float32),
    scratch_types=[
        pltpu.VMEM((_B_PER_W,), jnp.int32),
        pltpu.VMEM((_B_PER_W, EMB_DIM), jnp.float32),
        pltpu.SemaphoreType.DMA,
    ],
    compiler_params=pltpu.CompilerParams(use_tc_tiling_on_sc=False),
)
def _gather_kernel(idx_hbm, table_hbm, out_hbm, idx_v, rows_v, sem):
    wid = lax.axis_index("s") * _NC + lax.axis_index("c")
    base = wid * _B_PER_W
    pltpu.sync_copy(idx_hbm.at[pl.ds(base, _B_PER_W)], idx_v)
    pltpu.async_copy(table_hbm.at[idx_v], rows_v, sem).wait()
    pltpu.sync_copy(rows_v, out_hbm.at[pl.ds(base, _B_PER_W)])


def kernel(char_indices, table):
    return _gather_kernel(char_indices.astype(jnp.int32), table)


# speedup vs baseline: 2.2665x; 2.2665x over previous
import functools

import jax
import jax.numpy as jnp
from jax import lax
from jax.experimental import pallas as pl
from jax.experimental.pallas import tpu as pltpu
from jax.experimental.pallas import tpu_sc as plsc

NUM_EMB = 100000
EMB_DIM = 32
BATCH = 16384

_L = 16
_CHUNK = 4096


@functools.partial(
    pl.kernel,
    mesh=plsc.VectorSubcoreMesh(core_axis_name="c", subcore_axis_name="s"),
    out_type=jax.ShapeDtypeStruct((EMB_DIM, BATCH), jnp.float32),
    scratch_types=[
        pltpu.VMEM((NUM_EMB,), jnp.float32),
        pltpu.VMEM((BATCH,), jnp.int32),
        pltpu.VMEM((_CHUNK,), jnp.float32),
        pltpu.SemaphoreType.DMA,
        pltpu.SemaphoreType.DMA,
    ],
    compiler_params=pltpu.CompilerParams(needs_layout_passes=False),
)
def _gather_kernel(idx_hbm, tab_hbm, out_hbm, row_v, idx_v, out_v, sem_r, sem_i):
    j = lax.axis_index("s") * 2 + lax.axis_index("c")
    row_cp = pltpu.async_copy(tab_hbm.at[j], row_v, sem_r)
    idx_cp = pltpu.async_copy(idx_hbm, idx_v, sem_i)
    row_cp.wait()
    idx_cp.wait()

    def chunk_body(c, _):
        def gather_body(g, _):
            ivec = idx_v[pl.ds(c * _CHUNK + g * _L, _L)]
            out_v[pl.ds(g * _L, _L)] = plsc.load_gather(row_v, [ivec])
            return _

        lax.fori_loop(0, _CHUNK // _L, gather_body, 0, unroll=8)
        pltpu.sync_copy(out_v, out_hbm.at[j, pl.ds(c * _CHUNK, _CHUNK)])
        return _

    lax.fori_loop(0, BATCH // _CHUNK, chunk_body, 0)


def kernel(char_indices, table):
    out_t = _gather_kernel(char_indices.astype(jnp.int32), table.T)
    return out_t.T


# speedup vs baseline: 2.8652x; 1.2642x over previous
import functools

import jax
import jax.numpy as jnp
from jax import lax
from jax.experimental import pallas as pl
from jax.experimental.pallas import tpu as pltpu
from jax.experimental.pallas import tpu_sc as plsc

NUM_EMB = 100000
EMB_DIM = 32
BATCH = 16384

_L = 16
_CHUNK = 4096


@functools.partial(
    pl.kernel,
    mesh=plsc.VectorSubcoreMesh(core_axis_name="c", subcore_axis_name="s"),
    out_type=jax.ShapeDtypeStruct((EMB_DIM, BATCH), jnp.float32),
    scratch_types=[
        pltpu.VMEM((NUM_EMB,), jnp.float32),
        pltpu.VMEM((BATCH,), jnp.int32),
        pltpu.VMEM((2, _CHUNK), jnp.float32),
        pltpu.SemaphoreType.DMA,
        pltpu.SemaphoreType.DMA,
    ],
    compiler_params=pltpu.CompilerParams(needs_layout_passes=False),
)
def _gather_kernel(idx_hbm, tab_hbm, out_hbm, row_v, idx_v, out_v, sem_r, sem_w):
    j = lax.axis_index("s") * 2 + lax.axis_index("c")
    row_cp = pltpu.async_copy(tab_hbm.at[j], row_v, sem_r)
    idx_cp = pltpu.async_copy(idx_hbm, idx_v, sem_w)
    row_cp.wait()
    idx_cp.wait()

    write_cps = []
    for c in range(BATCH // _CHUNK):
        buf = c % 2
        if c >= 2:
            write_cps[c - 2].wait()

        @plsc.parallel_loop(0, _CHUNK // _L, unroll=8)
        def gather_body(g):
            ivec = idx_v[pl.ds(c * _CHUNK + g * _L, _L)]
            out_v[buf, pl.ds(g * _L, _L)] = plsc.load_gather(row_v, [ivec])

        write_cps.append(
            pltpu.async_copy(
                out_v.at[buf], out_hbm.at[j, pl.ds(c * _CHUNK, _CHUNK)], sem_w
            )
        )
    write_cps[-2].wait()
    write_cps[-1].wait()


def kernel(char_indices, table):
    out_t = _gather_kernel(char_indices.astype(jnp.int32), table.T)
    return out_t.T
